# baseline (speedup 1.0000x reference)
import functools
import math

import jax
import jax.numpy as jnp
from jax import lax
from jax.experimental import pallas as pl
from jax.experimental.pallas import tpu as pltpu
from jax.experimental.pallas import tpu_sc as plsc

D_MODEL = 128
SCALE = math.sqrt(D_MODEL)

_info = plsc.get_sparse_core_info()
NC, NS, L = _info.num_cores, _info.num_subcores, _info.num_lanes
NW = NC * NS

C = 128
NBUF = 3


def _make_sc_lookup(S: int, T: int, D: int):
    assert T == NW * C
    n_chunks = S
    n_steady = (n_chunks - 2 * NBUF) // NBUF

    mesh = plsc.VectorSubcoreMesh(core_axis_name="c", subcore_axis_name="s")

    @functools.partial(
        pl.kernel,
        mesh=mesh,
        out_type=jax.ShapeDtypeStruct((S, T, D), jnp.float32),
        scratch_types=[
            pltpu.VMEM((S, C), jnp.int32),
            pltpu.VMEM((NBUF, C, D), jnp.float32),
            pltpu.VMEM((NBUF, C, D), jnp.float32),
        ] + [pltpu.SemaphoreType.DMA] * (2 * NBUF),
    )
    def lookup(idx_hbm, table_hbm, out_hbm, idx_v, gbufs, sbufs, *sems):
        gsems, ssems = sems[:NBUF], sems[NBUF:]
        wid = lax.axis_index("s") * NC + lax.axis_index("c")
        tok0 = wid * C
        pltpu.sync_copy(idx_hbm.at[:, pl.ds(tok0, C)], idx_v)

        def g_start(j, b):
            pltpu.make_async_copy(
                table_hbm.at[idx_v.at[j]], gbufs.at[b], gsems[b]).start()

        def g_wait(b):
            pltpu.make_async_copy(
                table_hbm.at[idx_v.at[0]], gbufs.at[b], gsems[b]).wait()

        def s_start(j, b):
            pltpu.make_async_copy(
                sbufs.at[b], out_hbm.at[j, pl.ds(tok0, C)], ssems[b]).start()

        def s_wait(b):
            pltpu.make_async_copy(
                sbufs.at[b], out_hbm.at[0, pl.ds(tok0, C)], ssems[b]).wait()

        def scale_chunk(b):
            def rows(r2, carry):
                for dr in range(2):
                    for c8 in range(D // L):
                        sl = pl.ds(c8 * L, L)
                        sbufs[b, 2 * r2 + dr, sl] = (
                            gbufs[b, 2 * r2 + dr, sl] * SCALE)
                return carry
            lax.fori_loop(0, C // 2, rows, 0)

        for j in range(NBUF):
            g_start(j, j)

        for b in range(NBUF):
            g_wait(b)
            scale_chunk(b)
            s_start(b, b)
            g_start(b + NBUF, b)

        def outer(g, carry):
            for b in range(NBUF):
                j = NBUF * g + b
                g_wait(b)
                s_wait(b)
                scale_chunk(b)
                s_start(j, b)
                g_start(j + NBUF, b)
            return carry
        lax.fori_loop(1, 1 + n_steady, outer, 0)

        for j in range(NBUF + n_steady * NBUF, n_chunks):
            b = j % NBUF
            g_wait(b)
            s_wait(b)
            scale_chunk(b)
            s_start(j, b)
            if j + NBUF < n_chunks:
                g_start(j + NBUF, b)

        for j in range(n_chunks - NBUF, n_chunks):
            s_wait(j % NBUF)

    return lookup


def kernel(x, lut_weight):
    n_tok, seq = x.shape
    vocab, d = lut_weight.shape
    xt = jnp.swapaxes(x, 0, 1).astype(jnp.int32)
    out = _make_sc_lookup(seq, n_tok, d)(xt, lut_weight)
    return jnp.transpose(out, (1, 0, 2))


# speedup vs baseline: 1.0097x; 1.0097x over previous
import functools
import math

import jax
import jax.numpy as jnp
from jax import lax
from jax.experimental import pallas as pl
from jax.experimental.pallas import tpu as pltpu
from jax.experimental.pallas import tpu_sc as plsc

D_MODEL = 128
SCALE = math.sqrt(D_MODEL)

_info = plsc.get_sparse_core_info()
NC, NS, L = _info.num_cores, _info.num_subcores, _info.num_lanes
NW = NC * NS

C = 128
NBUF = 3


def _make_sc_lookup(S: int, T: int, D: int):
    assert T == NW * C
    n_chunks = S
    n_steady = (n_chunks - 2 * NBUF) // NBUF

    mesh = plsc.VectorSubcoreMesh(core_axis_name="c", subcore_axis_name="s")

    @functools.partial(
        pl.kernel,
        mesh=mesh,
        out_type=jax.ShapeDtypeStruct((S, T, D), jnp.float32),
        scratch_types=[
            pltpu.VMEM((S, C), jnp.int32),
            pltpu.VMEM((NBUF, C, D), jnp.float32),
            pltpu.VMEM((NBUF, C, D), jnp.float32),
        ] + [pltpu.SemaphoreType.DMA] * (2 * NBUF),
    )
    def lookup(idx_hbm, table_hbm, out_hbm, idx_v, gbufs, sbufs, *sems):
        gsems, ssems = sems[:NBUF], sems[NBUF:]
        wid = lax.axis_index("s") * NC + lax.axis_index("c")
        tok0 = wid * C
        pltpu.sync_copy(idx_hbm.at[:, pl.ds(tok0, C)], idx_v)

        def g_start(j, b):
            pltpu.make_async_copy(
                table_hbm.at[idx_v.at[j]], gbufs.at[b], gsems[b]).start()

        def g_wait(b):
            pltpu.make_async_copy(
                table_hbm.at[idx_v.at[0]], gbufs.at[b], gsems[b]).wait()

        def s_start(j, b):
            pltpu.make_async_copy(
                sbufs.at[b], out_hbm.at[j, pl.ds(tok0, C)], ssems[b]).start()

        def s_wait(b):
            pltpu.make_async_copy(
                sbufs.at[b], out_hbm.at[0, pl.ds(tok0, C)], ssems[b]).wait()

        def scale_chunk(b):
            def row(r, carry):
                for c8 in range(D // L):
                    sl = pl.ds(c8 * L, L)
                    sbufs[b, r, sl] = gbufs[b, r, sl] * SCALE
                return carry
            lax.fori_loop(0, C, row, 0)

        for j in range(NBUF):
            g_start(j, j)

        for b in range(NBUF):
            g_wait(b)
            scale_chunk(b)
            s_start(b, b)
            g_start(b + NBUF, b)

        def outer(g, carry):
            for b in range(NBUF):
                j = NBUF * g + b
                g_wait(b)
                s_wait(b)
                scale_chunk(b)
                g_start(j + NBUF, b)
                s_start(j, b)
            return carry
        lax.fori_loop(1, 1 + n_steady, outer, 0)

        for j in range(NBUF + n_steady * NBUF, n_chunks):
            b = j % NBUF
            g_wait(b)
            s_wait(b)
            scale_chunk(b)
            s_start(j, b)
            if j + NBUF < n_chunks:
                g_start(j + NBUF, b)

        for j in range(n_chunks - NBUF, n_chunks):
            s_wait(j % NBUF)

    return lookup


def kernel(x, lut_weight):
    n_tok, seq = x.shape
    vocab, d = lut_weight.shape
    xt = jnp.swapaxes(x, 0, 1).astype(jnp.int32)
    out = _make_sc_lookup(seq, n_tok, d)(xt, lut_weight)
    return jnp.transpose(out, (1, 0, 2))


# speedup vs baseline: 1.0134x; 1.0037x over previous
import functools
import math

import jax
import jax.numpy as jnp
from jax import lax
from jax.experimental import pallas as pl
from jax.experimental.pallas import tpu as pltpu
from jax.experimental.pallas import tpu_sc as plsc

D_MODEL = 128
SCALE = math.sqrt(D_MODEL)

_info = plsc.get_sparse_core_info()
NC, NS, L = _info.num_cores, _info.num_subcores, _info.num_lanes
NW = NC * NS

C = 128
NBUF = 3


def _make_sc_lookup(S: int, T: int, D: int):
    assert T == NW * C
    n_chunks = S
    n_steady = (n_chunks - 2 * NBUF) // NBUF

    mesh = plsc.VectorSubcoreMesh(core_axis_name="c", subcore_axis_name="s")

    @functools.partial(
        pl.kernel,
        mesh=mesh,
        out_type=jax.ShapeDtypeStruct((S, T, D), jnp.float32),
        scratch_types=[
            pltpu.VMEM((S, C), jnp.int32),
            pltpu.VMEM((NBUF, C, D), jnp.float32),
            pltpu.VMEM((NBUF, C, D), jnp.float32),
        ] + [pltpu.SemaphoreType.DMA] * (2 * NBUF),
    )
    def lookup(idx_hbm, table_hbm, out_hbm, idx_v, gbufs, sbufs, *sems):
        gsems, ssems = sems[:NBUF], sems[NBUF:]
        wid = lax.axis_index("s") * NC + lax.axis_index("c")
        tok0 = wid * C
        pltpu.sync_copy(idx_hbm.at[pl.ds(0, 8), pl.ds(tok0, C)],
                        idx_v.at[pl.ds(0, 8)])

        def g_start(j, b):
            pltpu.make_async_copy(
                table_hbm.at[idx_v.at[j]], gbufs.at[b], gsems[b]).start()

        def g_wait(b):
            pltpu.make_async_copy(
                table_hbm.at[idx_v.at[0]], gbufs.at[b], gsems[b]).wait()

        def s_start(j, b):
            pltpu.make_async_copy(
                sbufs.at[b], out_hbm.at[j, pl.ds(tok0, C)], ssems[b]).start()

        def s_wait(b):
            pltpu.make_async_copy(
                sbufs.at[b], out_hbm.at[0, pl.ds(tok0, C)], ssems[b]).wait()

        def scale_chunk(b):
            def row(r, carry):
                for c8 in range(D // L):
                    sl = pl.ds(c8 * L, L)
                    sbufs[b, r, sl] = gbufs[b, r, sl] * SCALE
                return carry
            lax.fori_loop(0, C, row, 0)

        for j in range(NBUF):
            g_start(j, j)

        pltpu.sync_copy(idx_hbm.at[pl.ds(8, S - 8), pl.ds(tok0, C)],
                        idx_v.at[pl.ds(8, S - 8)])

        for b in range(NBUF):
            g_wait(b)
            scale_chunk(b)
            s_start(b, b)
            g_start(b + NBUF, b)

        def outer(g, carry):
            for b in range(NBUF):
                j = NBUF * g + b
                g_wait(b)
                s_wait(b)
                scale_chunk(b)
                g_start(j + NBUF, b)
                s_start(j, b)
            return carry
        lax.fori_loop(1, 1 + n_steady, outer, 0)

        for j in range(NBUF + n_steady * NBUF, n_chunks):
            b = j % NBUF
            g_wait(b)
            s_wait(b)
            scale_chunk(b)
            s_start(j, b)
            if j + NBUF < n_chunks:
                g_start(j + NBUF, b)

        for j in range(n_chunks - NBUF, n_chunks):
            s_wait(j % NBUF)

    return lookup


def kernel(x, lut_weight):
    n_tok, seq = x.shape
    vocab, d = lut_weight.shape
    xt = jnp.swapaxes(x, 0, 1).astype(jnp.int32)
    out = _make_sc_lookup(seq, n_tok, d)(xt, lut_weight)
    return jnp.transpose(out, (1, 0, 2))
